# baseline (speedup 1.0000x reference)
import functools

import jax
import jax.numpy as jnp
from jax import lax
from jax.experimental import pallas as pl
from jax.experimental.pallas import tpu as pltpu
from jax.experimental.pallas import tpu_sc as plsc

_DIM = 64


@functools.partial(
    pl.kernel,
    mesh=plsc.VectorSubcoreMesh(
        core_axis_name="c", subcore_axis_name="s", num_cores=1
    ),
    out_type=jax.ShapeDtypeStruct((_DIM,), jnp.float32),
    scratch_types=[
        pltpu.VMEM((_DIM,), jnp.float32),
    ],
)
def _lookup(flat_hbm, out_hbm, row_v):
    s = lax.axis_index("s")

    @pl.when(s == 0)
    def _():
        pltpu.sync_copy(flat_hbm.at[pl.ds(0, _DIM)], row_v)
        pltpu.sync_copy(row_v, out_hbm)


def kernel(ascii_value, position, embeddings):
    del ascii_value, position
    return _lookup(embeddings.reshape(-1)).reshape(1, _DIM)
